# Initial kernel scaffold; baseline (speedup 1.0000x reference)
import jax
import jax.numpy as jnp
from jax.experimental import pallas as pl


def kernel(x, edge_index, batch, emb, W1, b1, W2, b2):
    raise NotImplementedError("write your pallas kernel here")


# speedup vs baseline: 9.9074x; 9.9074x over previous
import functools

import jax
import jax.numpy as jnp
from jax import lax
from jax.experimental import pallas as pl
from jax.experimental.pallas import tpu as pltpu
from jax.experimental.pallas import tpu_sc as plsc

N_NODES = 50000
N_EDGES = 800000
N_GRAPHS = 64
EMBED = 64
HID1 = 128
HID2 = 64

NPAD = 53248
EPAD = 819200
ECH_T = 400
NCH_T = 26
NSTRIPE = NPAD // 16
Q = 16

_MESH = plsc.VectorSubcoreMesh(core_axis_name="c", subcore_axis_name="s")
_SC_PARAMS = pltpu.CompilerParams(use_tc_tiling_on_sc=False)


def _fill_row_offsets(buf, base, n):
    for k in range(n // 16):
        buf[pl.ds(k * 16, 16)] = (
            base + k * 16 + lax.broadcasted_iota(jnp.int32, (16,), 0))


_GROUPS = [(g * 128, 128) for g in range(ECH_T // 128)]
_TAIL = (ECH_T // 128) * 128
if ECH_T % 128:
    _GROUPS.append((_TAIL, ECH_T - _TAIL))


@functools.partial(
    pl.kernel,
    out_type=[
        jax.ShapeDtypeStruct((NPAD, EMBED), jnp.float32),
        jax.ShapeDtypeStruct((NPAD,), jnp.float32),
    ],
    mesh=_MESH,
    compiler_params=_SC_PARAMS,
    scratch_types=[
        pltpu.VMEM((128,), jnp.int32),
        pltpu.VMEM((16,), jnp.int32),
        pltpu.VMEM((128, 128), jnp.int32),
        pltpu.VMEM((16, 128), jnp.int32),
        pltpu.VMEM((32,), jnp.int32),
        pltpu.VMEM((32, 128), jnp.int32),
        pltpu.VMEM((128,), jnp.float32),
        pltpu.VMEM((128, EMBED), jnp.float32),
        pltpu.VMEM_SHARED((NPAD,), jnp.float32),
    ],
)
def _p1_deg_and_embed(x2d, dst2d, emb, znode, h0_out, deg_out,
                      off_v, offt_v, idxe_v, idxet_v, offn_v, idxn_v,
                      ones_v, rows_v, deg_sh):
    c = lax.axis_index("c")
    s = lax.axis_index("s")

    @pl.when(c == 0)
    def _():
        pltpu.sync_copy(znode, deg_sh.at[pl.ds(s * NSTRIPE, NSTRIPE)])
        for k in range(8):
            ones_v[pl.ds(k * 16, 16)] = jnp.full((16,), 1.0, jnp.float32)
        plsc.subcore_barrier()

        for goff, glen in _GROUPS:
            offb, idxb = (off_v, idxe_v) if glen == 128 else (offt_v, idxet_v)
            _fill_row_offsets(offb, s * ECH_T + goff, glen)
            pltpu.sync_copy(dst2d.at[offb], idxb)

            def body(j, carry, idxb=idxb):
                pltpu.sync_copy(ones_v, deg_sh.at[idxb.at[j]], add=True)
                return carry

            lax.fori_loop(0, glen, body, 0)
        plsc.subcore_barrier()
        pltpu.sync_copy(deg_sh.at[pl.ds(s * NSTRIPE, NSTRIPE)],
                        deg_out.at[pl.ds(s * NSTRIPE, NSTRIPE)])

    @pl.when(c == 1)
    def _():
        nrows = NPAD // 128
        for k in range(2):
            offn_v[pl.ds(k * 16, 16)] = jnp.minimum(
                s * NCH_T + k * 16
                + lax.broadcasted_iota(jnp.int32, (16,), 0), nrows - 1)
        pltpu.sync_copy(x2d.at[offn_v], idxn_v)

        def body(j, carry):
            pltpu.sync_copy(emb.at[idxn_v.at[j]], rows_v)
            pltpu.sync_copy(rows_v, h0_out.at[pl.ds((s * NCH_T + j) * 128, 128)])
            return carry

        lax.fori_loop(0, NCH_T, body, 0)


@functools.partial(
    pl.kernel,
    out_type=[jax.ShapeDtypeStruct((NPAD, Q), jnp.float32)
              for _ in range(4)],
    mesh=_MESH,
    compiler_params=_SC_PARAMS,
    scratch_types=[
        pltpu.VMEM((128,), jnp.int32),
        pltpu.VMEM((16,), jnp.int32),
        pltpu.VMEM((128, 128), jnp.int32),
        pltpu.VMEM((128, 128), jnp.int32),
        pltpu.VMEM((16, 128), jnp.int32),
        pltpu.VMEM((16, 128), jnp.int32),
        pltpu.VMEM((128, Q), jnp.float32),
        pltpu.VMEM_SHARED((NPAD, Q), jnp.float32),
    ],
)
def _p_aggregate(hs0, hs1, hs2, hs3, src2d, dst2d, zrows,
                 agg0_out, agg1_out, agg2_out, agg3_out,
                 off_v, offt_v, idxs_v, idxd_v, idxst_v, idxdt_v,
                 rows_v, acc_sh):
    c = lax.axis_index("c")
    s = lax.axis_index("s")

    def quarter_pass(hs, agg_out):
        pltpu.sync_copy(zrows, acc_sh.at[pl.ds(s * NSTRIPE, NSTRIPE)])
        plsc.subcore_barrier()
        for goff, glen in _GROUPS:
            if glen == 128:
                offb, isb, idb = off_v, idxs_v, idxd_v
            else:
                offb, isb, idb = offt_v, idxst_v, idxdt_v
            _fill_row_offsets(offb, s * ECH_T + goff, glen)
            pltpu.sync_copy(src2d.at[offb], isb)
            pltpu.sync_copy(dst2d.at[offb], idb)

            def body(j, carry, isb=isb, idb=idb):
                pltpu.sync_copy(hs.at[isb.at[j]], rows_v)
                pltpu.sync_copy(rows_v, acc_sh.at[idb.at[j]], add=True)
                return carry

            lax.fori_loop(0, glen, body, 0)
        plsc.subcore_barrier()
        pltpu.sync_copy(acc_sh.at[pl.ds(s * NSTRIPE, NSTRIPE)],
                        agg_out.at[pl.ds(s * NSTRIPE, NSTRIPE)])

    @pl.when(c == 0)
    def _():
        quarter_pass(hs0, agg0_out)
        quarter_pass(hs1, agg1_out)

    @pl.when(c == 1)
    def _():
        quarter_pass(hs2, agg2_out)
        quarter_pass(hs3, agg3_out)


def _p2_body(deg_ref, h0_ref, dinv_ref, *hs_refs):
    dinv = lax.rsqrt(deg_ref[...] + 1.0)
    dinv_ref[...] = dinv
    hs = h0_ref[...] * dinv
    for q in range(4):
        hs_refs[q][...] = hs[:, q * Q:(q + 1) * Q]


def _p2_call(deg, h0):
    B = 1024
    grid = NPAD // B
    return pl.pallas_call(
        _p2_body,
        grid=(grid,),
        in_specs=[
            pl.BlockSpec((B, 1), lambda i: (i, 0)),
            pl.BlockSpec((B, EMBED), lambda i: (i, 0)),
        ],
        out_specs=[pl.BlockSpec((B, 1), lambda i: (i, 0))]
        + [pl.BlockSpec((B, Q), lambda i: (i, 0)) for _ in range(4)],
        out_shape=[jax.ShapeDtypeStruct((NPAD, 1), jnp.float32)]
        + [jax.ShapeDtypeStruct((NPAD, Q), jnp.float32) for _ in range(4)],
    )(deg, h0)


def _p4_body(dinv_ref, a0, a1, a2, a3, s0, s1, s2, s3,
             W1_ref, b1_ref, W2_ref, o0, o1, o2, o3):
    dinv = dinv_ref[...]
    h1 = b1_ref[...]
    for q, (a, ss) in enumerate(((a0, s0), (a1, s1), (a2, s2), (a3, s3))):
        xq = (a[...] + ss[...]) * dinv
        h1 = h1 + jnp.dot(xq, W1_ref[q * Q:(q + 1) * Q, :],
                          preferred_element_type=jnp.float32)
    h1 = jnp.maximum(h1, 0.0)
    z = jnp.dot(h1, W2_ref[...], preferred_element_type=jnp.float32) * dinv
    for q, o in enumerate((o0, o1, o2, o3)):
        o[...] = z[:, q * Q:(q + 1) * Q]


def _p4_call(dinv, aggs, hss, W1, b1r, W2):
    B = 1024
    grid = NPAD // B
    return pl.pallas_call(
        _p4_body,
        grid=(grid,),
        in_specs=[pl.BlockSpec((B, 1), lambda i: (i, 0))]
        + [pl.BlockSpec((B, Q), lambda i: (i, 0)) for _ in range(8)]
        + [
            pl.BlockSpec((EMBED, HID1), lambda i: (0, 0)),
            pl.BlockSpec((1, HID1), lambda i: (0, 0)),
            pl.BlockSpec((HID1, HID2), lambda i: (0, 0)),
        ],
        out_specs=[pl.BlockSpec((B, Q), lambda i: (i, 0)) for _ in range(4)],
        out_shape=[jax.ShapeDtypeStruct((NPAD, Q), jnp.float32)
                   for _ in range(4)],
    )(dinv, *aggs, *hss, W1, b1r, W2)


def _p6_body(dinv_ref, a0, a1, a2, a3, s0, s1, s2, s3,
             b2_ref, batch_ref, out_ref, acc_ref, cnt_ref):
    i = pl.program_id(0)
    nblk = pl.num_programs(0)
    dinv = dinv_ref[...]
    parts = [(a[...] + ss[...]) * dinv
             for a, ss in ((a0, s0), (a1, s1), (a2, s2), (a3, s3))]
    h2 = jnp.maximum(jnp.concatenate(parts, axis=1) + b2_ref[...], 0.0)
    gids = batch_ref[...]
    oh_t = (lax.broadcasted_iota(jnp.int32, (N_GRAPHS, 1), 0)
            == gids).astype(jnp.float32)
    sblk = jnp.dot(oh_t, h2, preferred_element_type=jnp.float32)
    cblk = jnp.dot(oh_t, jnp.ones((h2.shape[0], 1), jnp.float32),
                   preferred_element_type=jnp.float32)

    @pl.when(i == 0)
    def _():
        acc_ref[...] = sblk
        cnt_ref[...] = cblk

    @pl.when(i > 0)
    def _():
        acc_ref[...] += sblk
        cnt_ref[...] += cblk

    @pl.when(i == nblk - 1)
    def _():
        out_ref[...] = acc_ref[...] / jnp.maximum(cnt_ref[...], 1.0)


def _p6_call(dinv, aggs, hss, b2r, batch2d):
    B = 1024
    grid = NPAD // B
    return pl.pallas_call(
        _p6_body,
        grid=(grid,),
        in_specs=[pl.BlockSpec((B, 1), lambda i: (i, 0))]
        + [pl.BlockSpec((B, Q), lambda i: (i, 0)) for _ in range(8)]
        + [
            pl.BlockSpec((1, HID2), lambda i: (0, 0)),
            pl.BlockSpec((1, B), lambda i: (0, i)),
        ],
        out_specs=pl.BlockSpec((N_GRAPHS, HID2), lambda i: (0, 0)),
        out_shape=jax.ShapeDtypeStruct((N_GRAPHS, HID2), jnp.float32),
        scratch_shapes=[
            pltpu.VMEM((N_GRAPHS, HID2), jnp.float32),
            pltpu.VMEM((N_GRAPHS, 1), jnp.float32),
        ],
    )(dinv, *aggs, *hss, b2r, batch2d)


def kernel(x, edge_index, batch, emb, W1, b1, W2, b2):
    x2d = jnp.pad(x[:, 0], (0, NPAD - N_NODES)).reshape(NPAD // 128, 128)
    src2d = jnp.pad(edge_index[0], (0, EPAD - N_EDGES)).reshape(EPAD // 128, 128)
    dst2d = jnp.pad(edge_index[1], (0, EPAD - N_EDGES),
                    constant_values=N_NODES).reshape(EPAD // 128, 128)
    znode = jnp.zeros((NSTRIPE,), jnp.float32)
    zrows = jnp.zeros((NSTRIPE, Q), jnp.float32)
    batch2d = jnp.pad(batch, (0, NPAD - N_NODES),
                      constant_values=N_GRAPHS).reshape(1, NPAD)
    b1r = b1.reshape(1, HID1)
    b2r = b2.reshape(1, HID2)

    h0, deg = _p1_deg_and_embed(x2d, dst2d, emb, znode)
    dinv, *hs0 = _p2_call(deg.reshape(NPAD, 1), h0)
    agg1 = _p_aggregate(*hs0, src2d, dst2d, zrows)
    hs1 = _p4_call(dinv, agg1, hs0, W1, b1r, W2)
    agg2 = _p_aggregate(*hs1, src2d, dst2d, zrows)
    return _p6_call(dinv, agg2, hs1, b2r, batch2d)
